# Initial kernel scaffold; baseline (speedup 1.0000x reference)
import jax
import jax.numpy as jnp
from jax.experimental import pallas as pl


def kernel(node_features, ref_node_features, edge_features, ref_edge_features, atom_type, edge_type, mask_to_nrme, mask_to_erme):
    raise NotImplementedError("write your pallas kernel here")


# speedup vs baseline: 2.4917x; 2.4917x over previous
import functools

import jax
import jax.numpy as jnp
from jax import lax
from jax.experimental import pallas as pl
from jax.experimental.pallas import tpu as pltpu
from jax.experimental.pallas import tpu_sc as plsc

F = 128
L = 16
NW = 32
CHUNK = 80

E_ROWS = 320000
N_PAD = 12800
ET_NUM = 16
AT_NUM = 4


def _zero_rows(ref, rows):
    z = jnp.zeros((L,), jnp.float32)
    for r in range(rows):
        for f in range(F // L):
            ref[r, pl.ds(f * L, L)] = z


def _accum_rows(feat_hbm, ref_hbm, ty_hbm, base, fbuf, rbuf, tbuf,
                acc_abs, acc_sq, cnt):
    pltpu.sync_copy(feat_hbm.at[pl.ds(base, CHUNK)], fbuf)
    pltpu.sync_copy(ref_hbm.at[pl.ds(base, CHUNK)], rbuf)
    pltpu.sync_copy(ty_hbm.at[pl.ds(base, CHUNK)], tbuf.at[pl.ds(0, CHUNK)])

    lane = lax.broadcasted_iota(jnp.int32, (L,), 0)

    def row_body(i, c):
        t_vec = plsc.load_gather(tbuf, [jnp.full((L,), i, jnp.int32)])
        c = c + jnp.where(lane == t_vec, 1.0, 0.0)
        for f in range(F // L):
            e = fbuf[i, pl.ds(f * L, L)]
            r = rbuf[i, pl.ds(f * L, L)]
            d = e - r
            col = lane + (f * L)
            plsc.addupdate_scatter(acc_abs, [t_vec, col], jnp.abs(d))
            plsc.addupdate_scatter(acc_sq, [t_vec, col], d * d)
        return c

    return lax.fori_loop(0, CHUNK, row_body, cnt)


def _broadcast_counts(cnt_vec, cnt_vmem, cntb, rows):
    cnt_vmem[pl.ds(0, L)] = cnt_vec
    for r in range(rows):
        v = plsc.load_gather(cnt_vmem, [jnp.full((L,), r, jnp.int32)])
        for f in range(F // L):
            cntb[r, pl.ds(f * L, L)] = v


def _sc_partials(edge, ref_edge, et, node, ref_node, at):
    mesh = plsc.VectorSubcoreMesh(core_axis_name="c", subcore_axis_name="s")

    @functools.partial(
        pl.kernel,
        out_type=(
            jax.ShapeDtypeStruct((NW, ET_NUM, F), jnp.float32),
            jax.ShapeDtypeStruct((NW, ET_NUM, F), jnp.float32),
            jax.ShapeDtypeStruct((NW, ET_NUM, F), jnp.float32),
            jax.ShapeDtypeStruct((NW, AT_NUM, F), jnp.float32),
            jax.ShapeDtypeStruct((NW, AT_NUM, F), jnp.float32),
            jax.ShapeDtypeStruct((NW, AT_NUM, F), jnp.float32),
        ),
        mesh=mesh,
        compiler_params=pltpu.CompilerParams(needs_layout_passes=False),
        scratch_types=[
            pltpu.VMEM((CHUNK, F), jnp.float32),
            pltpu.VMEM((CHUNK, F), jnp.float32),
            pltpu.VMEM((128,), jnp.int32),
            pltpu.VMEM((ET_NUM, F), jnp.float32),
            pltpu.VMEM((ET_NUM, F), jnp.float32),
            pltpu.VMEM((8, F), jnp.float32),
            pltpu.VMEM((8, F), jnp.float32),
            pltpu.VMEM((L,), jnp.float32),
            pltpu.VMEM((ET_NUM, F), jnp.float32),
        ],
    )
    def sc(edge_h, refe_h, et_h, node_h, refn_h, at_h,
           out_ea, out_es, out_ec, out_na, out_ns, out_nc,
           fbuf, rbuf, tbuf, acc_ea, acc_es, acc_na, acc_ns, cnt_vmem, cntb):
        wid = lax.axis_index("s") * 2 + lax.axis_index("c")

        _zero_rows(acc_ea, ET_NUM)
        _zero_rows(acc_es, ET_NUM)
        _zero_rows(acc_na, 8)
        _zero_rows(acc_ns, 8)

        e_per_w = E_ROWS // NW
        n_per_w = N_PAD // NW

        def e_chunk(c, cnt):
            return _accum_rows(edge_h, refe_h, et_h, wid * e_per_w + c * CHUNK,
                               fbuf, rbuf, tbuf, acc_ea, acc_es, cnt)

        cnt_e = lax.fori_loop(0, e_per_w // CHUNK, e_chunk,
                              jnp.zeros((L,), jnp.float32))

        def n_chunk(c, cnt):
            return _accum_rows(node_h, refn_h, at_h, wid * n_per_w + c * CHUNK,
                               fbuf, rbuf, tbuf, acc_na, acc_ns, cnt)

        cnt_n = lax.fori_loop(0, n_per_w // CHUNK, n_chunk,
                              jnp.zeros((L,), jnp.float32))

        pltpu.sync_copy(acc_ea, out_ea.at[wid])
        pltpu.sync_copy(acc_es, out_es.at[wid])
        _broadcast_counts(cnt_e, cnt_vmem, cntb, ET_NUM)
        pltpu.sync_copy(cntb, out_ec.at[wid])
        pltpu.sync_copy(acc_na.at[pl.ds(0, AT_NUM)], out_na.at[wid])
        pltpu.sync_copy(acc_ns.at[pl.ds(0, AT_NUM)], out_ns.at[wid])
        _broadcast_counts(cnt_n, cnt_vmem, cntb, AT_NUM)
        pltpu.sync_copy(cntb.at[pl.ds(0, AT_NUM)], out_nc.at[wid])

    return sc(edge, ref_edge, et, node, ref_node, at)


def _combine_kernel(ea, es, ec, na, ns, nc, nmask, emask, out):
    def half_loss(s_abs, s_sq, cnt, mask):
        present = (cnt > 0.0).astype(jnp.float32)
        sel = present * mask
        denom = jnp.maximum(cnt, 1.0)
        ncnt = jnp.sum(sel)
        term_abs = jnp.sum(sel * s_abs / denom) / ncnt
        term_sq = jnp.sum(sel * s_sq / denom) / ncnt
        return 0.5 * (term_abs + jnp.sqrt(term_sq))

    hop = half_loss(jnp.sum(ea[...], axis=0), jnp.sum(es[...], axis=0),
                    jnp.sum(ec[...], axis=0), emask[...])
    ons = half_loss(jnp.sum(na[...], axis=0), jnp.sum(ns[...], axis=0),
                    jnp.sum(nc[...], axis=0), nmask[...])
    out[0, 0] = 0.5 * (ons + hop)


def kernel(node_features, ref_node_features, edge_features, ref_edge_features,
           atom_type, edge_type, mask_to_nrme, mask_to_erme):
    at = atom_type.astype(jnp.int32)
    et = edge_type.astype(jnp.int32)
    n = node_features.shape[0]
    at_pad = jnp.concatenate([at, jnp.full((N_PAD - n,), AT_NUM, jnp.int32)])
    zpad = jnp.zeros((N_PAD - n, F), jnp.float32)
    nf = jnp.concatenate([node_features, zpad])
    rnf = jnp.concatenate([ref_node_features, zpad])

    ea, es, ec, na, ns, nc = _sc_partials(edge_features, ref_edge_features, et,
                                          nf, rnf, at_pad)

    loss = pl.pallas_call(
        _combine_kernel,
        out_shape=jax.ShapeDtypeStruct((1, 1), jnp.float32),
        out_specs=pl.BlockSpec(memory_space=pltpu.SMEM),
    )(ea, es, ec, na, ns, nc,
      mask_to_nrme.astype(jnp.float32), mask_to_erme.astype(jnp.float32))
    return loss[0, 0]


# speedup vs baseline: 3.6667x; 1.4715x over previous
import functools

import jax
import jax.numpy as jnp
from jax import lax
from jax.experimental import pallas as pl
from jax.experimental.pallas import tpu as pltpu
from jax.experimental.pallas import tpu_sc as plsc

F = 128
L = 16
NW = 32
CHUNK = 80
GROUPS = CHUNK // L

E_ROWS = 320000
N_PAD = 12800
ET_NUM = 16
AT_NUM = 4


def _zero_rows(ref, rows):
    z = jnp.zeros((L,), jnp.float32)
    for r in range(rows):
        for f in range(F // L):
            ref[r, pl.ds(f * L, L)] = z


def _stream_accum(feat_hbm, ref_hbm, row0, nchunks, tloc,
                  fb, rb, sems, acc_abs, acc_sq, cnt0):
    lane = lax.broadcasted_iota(jnp.int32, (L,), 0)

    def start(c, slot):
        pltpu.async_copy(feat_hbm.at[pl.ds(row0 + c * CHUNK, CHUNK)],
                         fb.at[slot], sems.at[slot])
        pltpu.async_copy(ref_hbm.at[pl.ds(row0 + c * CHUNK, CHUNK)],
                         rb.at[slot], sems.at[slot])

    def wait(c, slot):
        pltpu.make_async_copy(feat_hbm.at[pl.ds(row0 + c * CHUNK, CHUNK)],
                              fb.at[slot], sems.at[slot]).wait()
        pltpu.make_async_copy(ref_hbm.at[pl.ds(row0 + c * CHUNK, CHUNK)],
                              rb.at[slot], sems.at[slot]).wait()

    def process(c, slot, cnt):
        def g_body(g, cnt):
            for rloc in range(L):
                lrow = c * CHUNK + g * L + rloc
                t_vec = plsc.load_gather(tloc, [jnp.full((L,), lrow, jnp.int32)])
                cnt = cnt + jnp.where(lane == t_vec, 1.0, 0.0)
                for f in range(F // L):
                    e = fb[slot, g * L + rloc, pl.ds(f * L, L)]
                    r = rb[slot, g * L + rloc, pl.ds(f * L, L)]
                    d = e - r
                    col = lane + (f * L)
                    plsc.addupdate_scatter(acc_abs, [t_vec, col], jnp.abs(d))
                    plsc.addupdate_scatter(acc_sq, [t_vec, col], d * d)
            return cnt

        return lax.fori_loop(0, GROUPS, g_body, cnt)

    start(0, 0)
    start(1, 1)

    def pair_body(cc, cnt):
        c0 = 2 * cc
        wait(c0, 0)
        cnt = process(c0, 0, cnt)
        pl.when(c0 + 2 < nchunks)(lambda: start(c0 + 2, 0))
        wait(c0 + 1, 1)
        cnt = process(c0 + 1, 1, cnt)
        pl.when(c0 + 3 < nchunks)(lambda: start(c0 + 3, 1))
        return cnt

    cnt = lax.fori_loop(0, nchunks // 2, pair_body, cnt0)
    wait(nchunks - 1, 0)
    return process(nchunks - 1, 0, cnt)


def _broadcast_counts(cnt_vec, cnt_vmem, cntb, rows):
    cnt_vmem[pl.ds(0, L)] = cnt_vec
    for r in range(rows):
        v = plsc.load_gather(cnt_vmem, [jnp.full((L,), r, jnp.int32)])
        for f in range(F // L):
            cntb[r, pl.ds(f * L, L)] = v


def _sc_partials(edge, ref_edge, et, node, ref_node, at):
    mesh = plsc.VectorSubcoreMesh(core_axis_name="c", subcore_axis_name="s")

    @functools.partial(
        pl.kernel,
        out_type=(
            jax.ShapeDtypeStruct((NW, ET_NUM, F), jnp.float32),
            jax.ShapeDtypeStruct((NW, ET_NUM, F), jnp.float32),
            jax.ShapeDtypeStruct((NW, ET_NUM, F), jnp.float32),
            jax.ShapeDtypeStruct((NW, AT_NUM, F), jnp.float32),
            jax.ShapeDtypeStruct((NW, AT_NUM, F), jnp.float32),
            jax.ShapeDtypeStruct((NW, AT_NUM, F), jnp.float32),
        ),
        mesh=mesh,
        compiler_params=pltpu.CompilerParams(needs_layout_passes=False),
        scratch_types=[
            pltpu.VMEM((2, CHUNK, F), jnp.float32),
            pltpu.VMEM((2, CHUNK, F), jnp.float32),
            pltpu.VMEM((E_ROWS // NW,), jnp.int32),
            pltpu.VMEM((N_PAD // NW,), jnp.int32),
            pltpu.VMEM((ET_NUM, F), jnp.float32),
            pltpu.VMEM((ET_NUM, F), jnp.float32),
            pltpu.VMEM((8, F), jnp.float32),
            pltpu.VMEM((8, F), jnp.float32),
            pltpu.VMEM((L,), jnp.float32),
            pltpu.VMEM((ET_NUM, F), jnp.float32),
            pltpu.SemaphoreType.DMA((2,)),
        ],
    )
    def sc(edge_h, refe_h, et_h, node_h, refn_h, at_h,
           out_ea, out_es, out_ec, out_na, out_ns, out_nc,
           fb, rb, tloc_e, tloc_n, acc_ea, acc_es, acc_na, acc_ns,
           cnt_vmem, cntb, sems):
        wid = lax.axis_index("s") * 2 + lax.axis_index("c")

        _zero_rows(acc_ea, ET_NUM)
        _zero_rows(acc_es, ET_NUM)
        _zero_rows(acc_na, 8)
        _zero_rows(acc_ns, 8)

        e_per_w = E_ROWS // NW
        n_per_w = N_PAD // NW
        pltpu.sync_copy(et_h.at[pl.ds(wid * e_per_w, e_per_w)], tloc_e)
        pltpu.sync_copy(at_h.at[pl.ds(wid * n_per_w, n_per_w)], tloc_n)

        cnt_e = _stream_accum(edge_h, refe_h, wid * e_per_w, e_per_w // CHUNK,
                              tloc_e, fb, rb, sems, acc_ea, acc_es,
                              jnp.zeros((L,), jnp.float32))
        cnt_n = _stream_accum(node_h, refn_h, wid * n_per_w, n_per_w // CHUNK,
                              tloc_n, fb, rb, sems, acc_na, acc_ns,
                              jnp.zeros((L,), jnp.float32))

        pltpu.sync_copy(acc_ea, out_ea.at[wid])
        pltpu.sync_copy(acc_es, out_es.at[wid])
        _broadcast_counts(cnt_e, cnt_vmem, cntb, ET_NUM)
        pltpu.sync_copy(cntb, out_ec.at[wid])
        pltpu.sync_copy(acc_na.at[pl.ds(0, AT_NUM)], out_na.at[wid])
        pltpu.sync_copy(acc_ns.at[pl.ds(0, AT_NUM)], out_ns.at[wid])
        _broadcast_counts(cnt_n, cnt_vmem, cntb, AT_NUM)
        pltpu.sync_copy(cntb.at[pl.ds(0, AT_NUM)], out_nc.at[wid])

    return sc(edge, ref_edge, et, node, ref_node, at)


def _combine_kernel(ea, es, ec, na, ns, nc, nmask, emask, out):
    def half_loss(s_abs, s_sq, cnt, mask):
        present = (cnt > 0.0).astype(jnp.float32)
        sel = present * mask
        denom = jnp.maximum(cnt, 1.0)
        ncnt = jnp.sum(sel)
        term_abs = jnp.sum(sel * s_abs / denom) / ncnt
        term_sq = jnp.sum(sel * s_sq / denom) / ncnt
        return 0.5 * (term_abs + jnp.sqrt(term_sq))

    hop = half_loss(jnp.sum(ea[...], axis=0), jnp.sum(es[...], axis=0),
                    jnp.sum(ec[...], axis=0), emask[...])
    ons = half_loss(jnp.sum(na[...], axis=0), jnp.sum(ns[...], axis=0),
                    jnp.sum(nc[...], axis=0), nmask[...])
    out[0, 0] = 0.5 * (ons + hop)


def kernel(node_features, ref_node_features, edge_features, ref_edge_features,
           atom_type, edge_type, mask_to_nrme, mask_to_erme):
    at = atom_type.astype(jnp.int32)
    et = edge_type.astype(jnp.int32)
    n = node_features.shape[0]
    at_pad = jnp.concatenate([at, jnp.full((N_PAD - n,), AT_NUM, jnp.int32)])
    zpad = jnp.zeros((N_PAD - n, F), jnp.float32)
    nf = jnp.concatenate([node_features, zpad])
    rnf = jnp.concatenate([ref_node_features, zpad])

    ea, es, ec, na, ns, nc = _sc_partials(edge_features, ref_edge_features, et,
                                          nf, rnf, at_pad)

    loss = pl.pallas_call(
        _combine_kernel,
        out_shape=jax.ShapeDtypeStruct((1, 1), jnp.float32),
        out_specs=pl.BlockSpec(memory_space=pltpu.SMEM),
    )(ea, es, ec, na, ns, nc,
      mask_to_nrme.astype(jnp.float32), mask_to_erme.astype(jnp.float32))
    return loss[0, 0]
